# Initial kernel scaffold; baseline (speedup 1.0000x reference)
import jax
import jax.numpy as jnp
from jax.experimental import pallas as pl


def kernel(regional_means, adj):
    raise NotImplementedError("write your pallas kernel here")


# speedup vs baseline: 2.6749x; 2.6749x over previous
import jax
import jax.numpy as jnp
from jax.experimental import pallas as pl

BLOCK_NUM = 256


def _sgc_kernel(rm_ref, adj_ref, out_ref):
    adj = adj_ref[0]
    a2 = jnp.dot(adj, adj, preferred_element_type=jnp.float32)
    a4 = jnp.dot(a2, a2, preferred_element_type=jnp.float32)
    row = jax.lax.broadcasted_iota(jnp.int32, (BLOCK_NUM, BLOCK_NUM), 0)
    col = jax.lax.broadcasted_iota(jnp.int32, (BLOCK_NUM, BLOCK_NUM), 1)
    a4 = jnp.where(row == col, 0.0, a4)
    out_ref[0] = jnp.dot(a4, rm_ref[0], preferred_element_type=jnp.float32)


def kernel(regional_means, adj):
    b, n, d = regional_means.shape
    return pl.pallas_call(
        _sgc_kernel,
        grid=(b,),
        in_specs=[
            pl.BlockSpec((1, n, d), lambda i: (i, 0, 0)),
            pl.BlockSpec((1, n, n), lambda i: (i, 0, 0)),
        ],
        out_specs=pl.BlockSpec((1, n, d), lambda i: (i, 0, 0)),
        out_shape=jax.ShapeDtypeStruct((b, n, d), jnp.float32),
    )(regional_means, adj)
